# baseline (speedup 1.0000x reference)
import functools

import jax
import jax.numpy as jnp
from jax import lax
from jax.experimental import pallas as pl
from jax.experimental.pallas import tpu as pltpu
from jax.experimental.pallas import tpu_sc as plsc

_HI = jax.lax.Precision.HIGHEST
_NT = (((1,), (1,)), ((), ()))
_NN = (((1,), (0,)), ((), ()))
_L = 16


def _b16(a):
    return a.astype(jnp.bfloat16).astype(jnp.float32)


def _score_body(p_ref, x_ref, y_ref):
    pb = p_ref[...]
    invn = jax.lax.rsqrt(jnp.maximum(jnp.sum(pb * pb), 1e-12))
    pnb = _b16(pb * invn)
    xb = _b16(x_ref[0])
    y_ref[0] = jnp.sum(xb * pnb, axis=0, keepdims=True)


def _sc_select_body(y_hbm, rank_hbm, sigy_hbm, y2_v, rank_v, sigy_v, *, nc, v):
    wid = lax.axis_index("s") * nc + lax.axis_index("c")
    pltpu.sync_copy(y_hbm.at[wid], y2_v.at[pl.ds(0, v)])
    pltpu.sync_copy(y_hbm.at[wid], y2_v.at[pl.ds(v, v)])

    ng = v // _L
    lane = lax.iota(jnp.int32, _L)
    one_i = jnp.ones((_L,), jnp.int32)
    zero_i = jnp.zeros((_L,), jnp.int32)

    def dbody(d, accs):
        d_vec = jnp.full((_L,), d, jnp.int32)
        out = []
        for i in range(ng):
            a = y2_v[pl.ds(i * _L, _L)]
            b = y2_v[pl.ds(i * _L + d, _L)]
            gtm = b > a
            eqm = b == a
            thr = jnp.full((_L,), v - _L * i, jnp.int32) - lane
            wrap = d_vec >= thr
            m = jnp.logical_or(gtm, jnp.logical_and(eqm, wrap))
            out.append(accs[i] + jnp.where(m, one_i, zero_i))
        return tuple(out)

    accs0 = tuple(jnp.zeros((_L,), jnp.int32) for _ in range(ng))
    accs = lax.fori_loop(1, v, dbody, dbody(0, accs0))
    one_f = jnp.ones((_L,), jnp.float32)
    for i in range(ng):
        rank_v[pl.ds(i * _L, _L)] = accs[i]
        a = y2_v[pl.ds(i * _L, _L)]
        sigy_v[pl.ds(i * _L, _L)] = one_f / (one_f + jnp.exp(jnp.zeros((_L,), jnp.float32) - a))

    pltpu.sync_copy(rank_v, rank_hbm.at[wid])
    pltpu.sync_copy(sigy_v, sigy_hbm.at[wid])


def _gather_body(rank_ref, sigy_ref, x_ref, a_ref, xo_ref, ao_ref, *, kk):
    v = rank_ref.shape[2]
    nk = a_ref.shape[1]
    rkrow = rank_ref[0]
    sgrow = sigy_ref[0]
    jj = jax.lax.broadcasted_iota(jnp.int32, (kk, v), 0)
    stf = (jj == rkrow).astype(jnp.float32)
    sgate = jax.lax.dot_general(sgrow, stf, _NT,
                                preferred_element_type=jnp.float32, precision=_HI)
    stb = stf.astype(jnp.bfloat16)

    xf = x_ref[0]
    xhi = xf.astype(jnp.bfloat16)
    xlo = (xf - xhi.astype(jnp.float32)).astype(jnp.bfloat16)
    acc = jax.lax.dot_general(xhi, stb, _NT, preferred_element_type=jnp.float32)
    acc += jax.lax.dot_general(xlo, stb, _NT, preferred_element_type=jnp.float32)
    xo_ref[0] = acc * sgate

    for k in range(nk):
        a = a_ref[0, k].astype(jnp.bfloat16)
        u = jax.lax.dot_general(stb, a, _NN,
                                preferred_element_type=jnp.float32)
        asel = jax.lax.dot_general(a, stb, _NT,
                                   preferred_element_type=jnp.float32)
        ao_ref[0, k] = jax.lax.dot_general(
            u.astype(jnp.bfloat16), asel.astype(jnp.bfloat16), _NN,
            preferred_element_type=jnp.float32)


@jax.jit
def kernel(x, A, p):
    n, c, t, v = x.shape
    ct = c * t
    kk = v // 2
    nk = A.shape[1]

    xr = x.reshape(n, ct, v)
    pc = p.reshape(ct, 1)

    y3 = pl.pallas_call(
        _score_body,
        grid=(n,),
        in_specs=[
            pl.BlockSpec((ct, 1), lambda i: (0, 0)),
            pl.BlockSpec((1, ct, v), lambda i: (i, 0, 0)),
        ],
        out_specs=pl.BlockSpec((1, 1, v), lambda i: (i, 0, 0)),
        out_shape=jax.ShapeDtypeStruct((n, 1, v), jnp.float32),
    )(pc, xr)

    info = plsc.get_sparse_core_info()
    nc = info.num_cores
    mesh = plsc.VectorSubcoreMesh(core_axis_name="c", subcore_axis_name="s")
    sc_select = pl.kernel(
        functools.partial(_sc_select_body, nc=nc, v=v),
        out_type=[
            jax.ShapeDtypeStruct((n, v), jnp.int32),
            jax.ShapeDtypeStruct((n, v), jnp.float32),
        ],
        mesh=mesh,
        scratch_types=[
            pltpu.VMEM((2 * v,), jnp.float32),
            pltpu.VMEM((v,), jnp.int32),
            pltpu.VMEM((v,), jnp.float32),
        ],
    )
    rank, sigy = sc_select(y3.reshape(n, v))
    rank3 = rank.reshape(n, 1, v)
    sigy3 = sigy.reshape(n, 1, v)

    xo, A_out = pl.pallas_call(
        functools.partial(_gather_body, kk=kk),
        grid=(n,),
        in_specs=[
            pl.BlockSpec((1, 1, v), lambda i: (i, 0, 0)),
            pl.BlockSpec((1, 1, v), lambda i: (i, 0, 0)),
            pl.BlockSpec((1, ct, v), lambda i: (i, 0, 0)),
            pl.BlockSpec((1, nk, v, v), lambda i: (i, 0, 0, 0)),
        ],
        out_specs=[
            pl.BlockSpec((1, ct, kk), lambda i: (i, 0, 0)),
            pl.BlockSpec((1, nk, kk, kk), lambda i: (i, 0, 0, 0)),
        ],
        out_shape=[
            jax.ShapeDtypeStruct((n, ct, kk), jnp.float32),
            jax.ShapeDtypeStruct((n, nk, kk, kk), jnp.float32),
        ],
    )(rank3, sigy3, xr, A)

    return xo.reshape(n, c, t, kk), A_out


# speedup vs baseline: 1.0224x; 1.0224x over previous
import functools

import jax
import jax.numpy as jnp
from jax import lax
from jax.experimental import pallas as pl
from jax.experimental.pallas import tpu as pltpu
from jax.experimental.pallas import tpu_sc as plsc

_HI = jax.lax.Precision.HIGHEST
_NT = (((1,), (1,)), ((), ()))
_NN = (((1,), (0,)), ((), ()))
_L = 16


def _b16(a):
    return a.astype(jnp.bfloat16).astype(jnp.float32)


def _score_body(p_ref, x_ref, y_ref):
    pb = p_ref[...]
    invn = jax.lax.rsqrt(jnp.maximum(jnp.sum(pb * pb), 1e-12))
    pnb = _b16(pb * invn)
    xb = _b16(x_ref[0])
    y_ref[0] = jnp.sum(xb * pnb, axis=0, keepdims=True)


def _sc_select_body(y_hbm, rank_hbm, sigy_hbm, y2_v, rank_v, sigy_v, *, nc, v):
    wid = lax.axis_index("s") * nc + lax.axis_index("c")
    pltpu.sync_copy(y_hbm.at[wid], y2_v.at[pl.ds(0, v)])
    pltpu.sync_copy(y_hbm.at[wid], y2_v.at[pl.ds(v, v)])

    ng = v // _L
    lane = lax.iota(jnp.int32, _L)
    one_i = jnp.ones((_L,), jnp.int32)
    zero_i = jnp.zeros((_L,), jnp.int32)
    unroll = 8

    def dbody(k, accs):
        out = list(accs)
        for s in range(unroll):
            d = k * unroll + s
            d_vec = jnp.full((_L,), d, jnp.int32)
            for i in range(ng):
                a = y2_v[pl.ds(i * _L, _L)]
                b = y2_v[pl.ds(i * _L + d, _L)]
                thr = jnp.full((_L,), v - _L * i, jnp.int32) - lane
                m = jnp.where(d_vec >= thr, b >= a, b > a)
                out[i] = out[i] + jnp.where(m, one_i, zero_i)
        return tuple(out)

    accs0 = tuple(jnp.zeros((_L,), jnp.int32) for _ in range(ng))
    accs = lax.fori_loop(0, v // unroll, dbody, accs0)
    one_f = jnp.ones((_L,), jnp.float32)
    for i in range(ng):
        rank_v[pl.ds(i * _L, _L)] = accs[i]
        a = y2_v[pl.ds(i * _L, _L)]
        sigy_v[pl.ds(i * _L, _L)] = one_f / (one_f + jnp.exp(jnp.zeros((_L,), jnp.float32) - a))

    pltpu.sync_copy(rank_v, rank_hbm.at[wid])
    pltpu.sync_copy(sigy_v, sigy_hbm.at[wid])


def _gather_body(rank_ref, sigy_ref, x_ref, a_ref, xo_ref, ao_ref, *, kk):
    v = rank_ref.shape[2]
    nk = a_ref.shape[1]
    rkrow = rank_ref[0]
    sgrow = sigy_ref[0]
    jj = jax.lax.broadcasted_iota(jnp.int32, (kk, v), 0)
    stf = (jj == rkrow).astype(jnp.float32)
    sgate = jax.lax.dot_general(sgrow, stf, _NT,
                                preferred_element_type=jnp.float32, precision=_HI)
    stb = stf.astype(jnp.bfloat16)

    xf = x_ref[0]
    xhi = xf.astype(jnp.bfloat16)
    xlo = (xf - xhi.astype(jnp.float32)).astype(jnp.bfloat16)
    acc = jax.lax.dot_general(xhi, stb, _NT, preferred_element_type=jnp.float32)
    acc += jax.lax.dot_general(xlo, stb, _NT, preferred_element_type=jnp.float32)
    xo_ref[0] = acc * sgate

    for k in range(nk):
        a = a_ref[0, k].astype(jnp.bfloat16)
        u = jax.lax.dot_general(stb, a, _NN,
                                preferred_element_type=jnp.float32)
        asel = jax.lax.dot_general(a, stb, _NT,
                                   preferred_element_type=jnp.float32)
        ao_ref[0, k] = jax.lax.dot_general(
            u.astype(jnp.bfloat16), asel.astype(jnp.bfloat16), _NN,
            preferred_element_type=jnp.float32)


@jax.jit
def kernel(x, A, p):
    n, c, t, v = x.shape
    ct = c * t
    kk = v // 2
    nk = A.shape[1]

    xr = x.reshape(n, ct, v)
    pc = p.reshape(ct, 1)

    y3 = pl.pallas_call(
        _score_body,
        grid=(n,),
        in_specs=[
            pl.BlockSpec((ct, 1), lambda i: (0, 0)),
            pl.BlockSpec((1, ct, v), lambda i: (i, 0, 0)),
        ],
        out_specs=pl.BlockSpec((1, 1, v), lambda i: (i, 0, 0)),
        out_shape=jax.ShapeDtypeStruct((n, 1, v), jnp.float32),
    )(pc, xr)

    info = plsc.get_sparse_core_info()
    nc = info.num_cores
    mesh = plsc.VectorSubcoreMesh(core_axis_name="c", subcore_axis_name="s")
    sc_select = pl.kernel(
        functools.partial(_sc_select_body, nc=nc, v=v),
        out_type=[
            jax.ShapeDtypeStruct((n, v), jnp.int32),
            jax.ShapeDtypeStruct((n, v), jnp.float32),
        ],
        mesh=mesh,
        scratch_types=[
            pltpu.VMEM((2 * v,), jnp.float32),
            pltpu.VMEM((v,), jnp.int32),
            pltpu.VMEM((v,), jnp.float32),
        ],
    )
    rank, sigy = sc_select(y3.reshape(n, v))
    rank3 = rank.reshape(n, 1, v)
    sigy3 = sigy.reshape(n, 1, v)

    xo, A_out = pl.pallas_call(
        functools.partial(_gather_body, kk=kk),
        grid=(n,),
        in_specs=[
            pl.BlockSpec((1, 1, v), lambda i: (i, 0, 0)),
            pl.BlockSpec((1, 1, v), lambda i: (i, 0, 0)),
            pl.BlockSpec((1, ct, v), lambda i: (i, 0, 0)),
            pl.BlockSpec((1, nk, v, v), lambda i: (i, 0, 0, 0)),
        ],
        out_specs=[
            pl.BlockSpec((1, ct, kk), lambda i: (i, 0, 0)),
            pl.BlockSpec((1, nk, kk, kk), lambda i: (i, 0, 0, 0)),
        ],
        out_shape=[
            jax.ShapeDtypeStruct((n, ct, kk), jnp.float32),
            jax.ShapeDtypeStruct((n, nk, kk, kk), jnp.float32),
        ],
    )(rank3, sigy3, xr, A)

    return xo.reshape(n, c, t, kk), A_out
